# baseline (speedup 1.0000x reference)
import functools

import numpy as np
import jax
import jax.numpy as jnp
from jax.experimental import pallas as pl
from jax.experimental.pallas import tpu as pltpu

_RADIUS = 1.0
_GROUPS = 4
_BI = 2048


def _leaky(x):
    return jnp.where(x >= 0, x, 0.2 * x)


def _group_norm(x, gamma_row, beta_row, groups, eps=1e-5):
    n, c = x.shape
    cg = c // groups
    ones = jnp.ones((1, n), jnp.float32)
    s = jnp.dot(ones, x, preferred_element_type=jnp.float32)
    ss = jnp.dot(ones, x * x, preferred_element_type=jnp.float32)
    a_parts, b_parts = [], []
    for g in range(groups):
        sl = slice(g * cg, (g + 1) * cg)
        m = jnp.sum(s[0:1, sl]) / (n * cg)
        var = jnp.sum(ss[0:1, sl]) / (n * cg) - m * m
        inv = 1.0 / jnp.sqrt(var + eps)
        a_parts.append(gamma_row[0:1, sl] * inv)
        b_parts.append(beta_row[0:1, sl] - gamma_row[0:1, sl] * (m * inv))
    a_row = jnp.concatenate(a_parts, axis=1)
    b_row = jnp.concatenate(b_parts, axis=1)
    return x * a_row + b_row


def _body(pts_c_ref, pts_r_ref, nrm_c_ref, nrm_r_ref, nuv9_ref,
          feat_ref, w1t_ref, b1_ref, w2t_ref, b2_ref, gin_ref, bein_ref,
          a1_ref, b1c_ref, a2t_ref, b2r_ref,
          wo1t_ref, bo1_ref, wo2t_ref, bo2_ref, gout_ref, beout_ref,
          out_ref, f_ref, z_ref, *, n_steps, bi, cuts):
    i = pl.program_id(0)

    @pl.when(i == 0)
    def _compute_f():
        h = _leaky(jnp.dot(feat_ref[...], w1t_ref[...],
                           preferred_element_type=jnp.float32) + b1_ref[...])
        h = _leaky(jnp.dot(h, w2t_ref[...],
                           preferred_element_type=jnp.float32) + b2_ref[...])
        f_ref[...] = _group_norm(h, gin_ref[...], bein_ref[...], _GROUPS)

    scale = 1.0 / (np.sqrt(2.0) * _RADIUS)
    ds = pl.ds(i * bi, bi)

    xs_i = pts_c_ref[ds, :] * scale
    xs_jt = pts_r_ref[...] * scale
    n_i = nrm_c_ref[ds, :]

    n = pts_r_ref.shape[1]
    ones_col = jnp.ones((bi, 1), jnp.float32)
    ones_row = jnp.ones((1, n), jnp.float32)
    r2_col = jnp.sum(xs_i * xs_i, axis=1, keepdims=True)
    r2_row = jnp.sum(xs_jt * xs_jt, axis=0, keepdims=True)

    lhs_d2 = jnp.concatenate([xs_i * -2.0, r2_col, ones_col], axis=1)
    rhs_d2 = jnp.concatenate([xs_jt, ones_row, r2_row], axis=0)
    d2 = jnp.dot(lhs_d2, rhs_d2, preferred_element_type=jnp.float32)
    lhs_t = jnp.concatenate([-n_i, 2.0 * ones_col], axis=1)
    rhs_t = jnp.concatenate([nrm_r_ref[...], ones_row], axis=0)
    t = jnp.dot(lhs_t, rhs_t, preferred_element_type=jnp.float32)

    w = jnp.exp(-(d2 * t) * t)

    nuv_i = nuv9_ref[ds, :]
    rhs_l = jnp.concatenate([xs_jt, ones_row], axis=0)
    f = f_ref[...]
    acc = jnp.dot(w, f, preferred_element_type=jnp.float32) * b2r_ref[...]
    for c in range(cuts):
        mc = (a1_ref[c, 0] * nuv_i[:, 0:3] + a1_ref[c, 1] * nuv_i[:, 3:6]
              + a1_ref[c, 2] * nuv_i[:, 6:9])
        col_c = b1c_ref[c] - jnp.sum(mc * xs_i, axis=1, keepdims=True)
        lc = jnp.dot(jnp.concatenate([mc, col_c], axis=1), rhs_l,
                     preferred_element_type=jnp.float32)
        acc = acc + jnp.dot(w * jnp.maximum(lc, 0.0), f,
                            preferred_element_type=jnp.float32) * a2t_ref[c:c + 1, :]

    z = _leaky(jnp.dot(acc, wo1t_ref[...],
                       preferred_element_type=jnp.float32) + bo1_ref[...])
    z = _leaky(jnp.dot(z, wo2t_ref[...],
                       preferred_element_type=jnp.float32) + bo2_ref[...])
    z_ref[ds, :] = z

    @pl.when(i == n_steps - 1)
    def _finish():
        out_ref[...] = _group_norm(z_ref[...], gout_ref[...], beout_ref[...],
                                   _GROUPS)


def kernel(points, nuv, features, W1, b1, W2, b2, g_in, be_in, A1, B1,
           A2, B2, Wo1, bo1, Wo2, bo2, g_out, be_out):
    n = points.shape[0]
    h = W1.shape[0]
    o = Wo1.shape[0]
    cuts = A1.shape[0]
    n_steps = n // _BI

    normals = nuv[:, 0, :]
    args = (
        points, points.T, normals, normals.T, nuv.reshape(n, 9),
        features, W1.T, b1[None], W2.T, b2[None], g_in[None], be_in[None],
        A1, B1, A2.T, B2[None],
        Wo1.T, bo1[None], Wo2.T, bo2[None], g_out[None], be_out[None],
    )

    def full_spec(a):
        nd = a.ndim
        return pl.BlockSpec(a.shape, lambda i, _nd=nd: (0,) * _nd)

    in_specs = [full_spec(a) for a in args]
    in_specs[12] = pl.BlockSpec(memory_space=pltpu.SMEM)
    in_specs[13] = pl.BlockSpec(memory_space=pltpu.SMEM)

    body = functools.partial(_body, n_steps=n_steps, bi=_BI, cuts=cuts)
    return pl.pallas_call(
        body,
        grid=(n_steps,),
        in_specs=in_specs,
        out_specs=pl.BlockSpec((n, o), lambda i: (0, 0)),
        out_shape=jax.ShapeDtypeStruct((n, o), jnp.float32),
        scratch_shapes=[
            pltpu.VMEM((n, h), jnp.float32),
            pltpu.VMEM((n, o), jnp.float32),
        ],
        compiler_params=pltpu.CompilerParams(
            dimension_semantics=("arbitrary",)),
    )(*args)


# speedup vs baseline: 1.0042x; 1.0042x over previous
import functools

import numpy as np
import jax
import jax.numpy as jnp
from jax.experimental import pallas as pl
from jax.experimental.pallas import tpu as pltpu

_RADIUS = 1.0
_GROUPS = 4
_BI = 1024


def _leaky(x):
    return jnp.where(x >= 0, x, 0.2 * x)


def _group_norm(x, gamma_row, beta_row, groups, eps=1e-5):
    n, c = x.shape
    cg = c // groups
    ones = jnp.ones((1, n), jnp.float32)
    s = jnp.dot(ones, x, preferred_element_type=jnp.float32)
    ss = jnp.dot(ones, x * x, preferred_element_type=jnp.float32)
    a_parts, b_parts = [], []
    for g in range(groups):
        sl = slice(g * cg, (g + 1) * cg)
        m = jnp.sum(s[0:1, sl]) / (n * cg)
        var = jnp.sum(ss[0:1, sl]) / (n * cg) - m * m
        inv = 1.0 / jnp.sqrt(var + eps)
        a_parts.append(gamma_row[0:1, sl] * inv)
        b_parts.append(beta_row[0:1, sl] - gamma_row[0:1, sl] * (m * inv))
    a_row = jnp.concatenate(a_parts, axis=1)
    b_row = jnp.concatenate(b_parts, axis=1)
    return x * a_row + b_row


def _body(pts_c_ref, pts_r_ref, nrm_c_ref, nrm_r_ref, nuv9_ref,
          feat_ref, w1t_ref, b1_ref, w2t_ref, b2_ref, gin_ref, bein_ref,
          a1_ref, b1c_ref, a2t_ref, b2r_ref,
          wo1t_ref, bo1_ref, wo2t_ref, bo2_ref, gout_ref, beout_ref,
          out_ref, f_ref, z_ref, *, n_steps, bi, cuts):
    i = pl.program_id(0)

    @pl.when(i == 0)
    def _compute_f():
        h = _leaky(jnp.dot(feat_ref[...], w1t_ref[...],
                           preferred_element_type=jnp.float32) + b1_ref[...])
        h = _leaky(jnp.dot(h, w2t_ref[...],
                           preferred_element_type=jnp.float32) + b2_ref[...])
        f_ref[...] = _group_norm(h, gin_ref[...], bein_ref[...],
                                 _GROUPS).astype(jnp.bfloat16)

    scale = 1.0 / (np.sqrt(2.0) * _RADIUS)
    ds = pl.ds(i * bi, bi)

    xs_i = pts_c_ref[ds, :] * scale
    xs_jt = pts_r_ref[...] * scale
    n_i = nrm_c_ref[ds, :]

    n = pts_r_ref.shape[1]
    ones_col = jnp.ones((bi, 1), jnp.float32)
    ones_row = jnp.ones((1, n), jnp.float32)
    r2_col = jnp.sum(xs_i * xs_i, axis=1, keepdims=True)
    r2_row = jnp.sum(xs_jt * xs_jt, axis=0, keepdims=True)

    lhs_d2 = jnp.concatenate([xs_i * -2.0, r2_col, ones_col], axis=1)
    rhs_d2 = jnp.concatenate([xs_jt, ones_row, r2_row], axis=0)
    d2 = jnp.dot(lhs_d2, rhs_d2, preferred_element_type=jnp.float32)
    lhs_t = jnp.concatenate([-n_i, 2.0 * ones_col], axis=1)
    rhs_t = jnp.concatenate([nrm_r_ref[...], ones_row], axis=0)
    t = jnp.dot(lhs_t, rhs_t, preferred_element_type=jnp.float32)

    w = jnp.exp(-(d2 * t) * t)

    nuv_i = nuv9_ref[ds, :]
    rhs_l = jnp.concatenate([xs_jt, ones_row], axis=0)
    f = f_ref[...]
    w16 = w.astype(jnp.bfloat16)
    acc = jnp.dot(w16, f, preferred_element_type=jnp.float32) * b2r_ref[...]
    for c in range(cuts):
        mc = (a1_ref[c, 0] * nuv_i[:, 0:3] + a1_ref[c, 1] * nuv_i[:, 3:6]
              + a1_ref[c, 2] * nuv_i[:, 6:9])
        col_c = b1c_ref[c] - jnp.sum(mc * xs_i, axis=1, keepdims=True)
        lc = jnp.dot(jnp.concatenate([mc, col_c], axis=1), rhs_l,
                     preferred_element_type=jnp.float32)
        whc = (w * jnp.maximum(lc, 0.0)).astype(jnp.bfloat16)
        acc = acc + jnp.dot(whc, f,
                            preferred_element_type=jnp.float32) * a2t_ref[c:c + 1, :]

    z = _leaky(jnp.dot(acc, wo1t_ref[...],
                       preferred_element_type=jnp.float32) + bo1_ref[...])
    z = _leaky(jnp.dot(z, wo2t_ref[...],
                       preferred_element_type=jnp.float32) + bo2_ref[...])
    z_ref[ds, :] = z

    @pl.when(i == n_steps - 1)
    def _finish():
        out_ref[...] = _group_norm(z_ref[...], gout_ref[...], beout_ref[...],
                                   _GROUPS)


def kernel(points, nuv, features, W1, b1, W2, b2, g_in, be_in, A1, B1,
           A2, B2, Wo1, bo1, Wo2, bo2, g_out, be_out):
    n = points.shape[0]
    h = W1.shape[0]
    o = Wo1.shape[0]
    cuts = A1.shape[0]
    n_steps = n // _BI

    normals = nuv[:, 0, :]
    args = (
        points, points.T, normals, normals.T, nuv.reshape(n, 9),
        features, W1.T, b1[None], W2.T, b2[None], g_in[None], be_in[None],
        A1, B1, A2.T, B2[None],
        Wo1.T, bo1[None], Wo2.T, bo2[None], g_out[None], be_out[None],
    )

    def full_spec(a):
        nd = a.ndim
        return pl.BlockSpec(a.shape, lambda i, _nd=nd: (0,) * _nd)

    in_specs = [full_spec(a) for a in args]
    in_specs[12] = pl.BlockSpec(memory_space=pltpu.SMEM)
    in_specs[13] = pl.BlockSpec(memory_space=pltpu.SMEM)

    body = functools.partial(_body, n_steps=n_steps, bi=_BI, cuts=cuts)
    return pl.pallas_call(
        body,
        grid=(n_steps,),
        in_specs=in_specs,
        out_specs=pl.BlockSpec((n, o), lambda i: (0, 0)),
        out_shape=jax.ShapeDtypeStruct((n, o), jnp.float32),
        scratch_shapes=[
            pltpu.VMEM((n, h), jnp.bfloat16),
            pltpu.VMEM((n, o), jnp.float32),
        ],
        compiler_params=pltpu.CompilerParams(
            dimension_semantics=("arbitrary",)),
    )(*args)


# speedup vs baseline: 1.0084x; 1.0042x over previous
import functools

import numpy as np
import jax
import jax.numpy as jnp
from jax.experimental import pallas as pl
from jax.experimental.pallas import tpu as pltpu

_RADIUS = 1.0
_GROUPS = 4
_BI = 1024


def _leaky(x):
    return jnp.where(x >= 0, x, 0.2 * x)


def _group_norm(x, gamma_row, beta_row, groups, eps=1e-5):
    n, c = x.shape
    cg = c // groups
    ones = jnp.ones((1, n), jnp.float32)
    s = jnp.dot(ones, x, preferred_element_type=jnp.float32)
    ss = jnp.dot(ones, x * x, preferred_element_type=jnp.float32)
    a_parts, b_parts = [], []
    for g in range(groups):
        sl = slice(g * cg, (g + 1) * cg)
        m = jnp.sum(s[0:1, sl]) / (n * cg)
        var = jnp.sum(ss[0:1, sl]) / (n * cg) - m * m
        inv = 1.0 / jnp.sqrt(var + eps)
        a_parts.append(gamma_row[0:1, sl] * inv)
        b_parts.append(beta_row[0:1, sl] - gamma_row[0:1, sl] * (m * inv))
    a_row = jnp.concatenate(a_parts, axis=1)
    b_row = jnp.concatenate(b_parts, axis=1)
    return x * a_row + b_row


def _body(pts_c_ref, pts_r_ref, nrm_c_ref, nrm_r_ref, nuv9_ref,
          feat_ref, w1t_ref, b1_ref, w2t_ref, b2_ref, gin_ref, bein_ref,
          a1_ref, b1c_ref, a2t_ref, b2r_ref,
          wo1t_ref, bo1_ref, wo2t_ref, bo2_ref, gout_ref, beout_ref,
          out_ref, f_ref, z_ref, *, n_steps, bi, cuts):
    i = pl.program_id(0)

    @pl.when(i == 0)
    def _compute_f():
        h = _leaky(jnp.dot(feat_ref[...], w1t_ref[...],
                           preferred_element_type=jnp.float32) + b1_ref[...])
        h = _leaky(jnp.dot(h, w2t_ref[...],
                           preferred_element_type=jnp.float32) + b2_ref[...])
        f_ref[...] = _group_norm(h, gin_ref[...], bein_ref[...], _GROUPS)

    scale = 1.0 / (np.sqrt(2.0) * _RADIUS)
    ds = pl.ds(i * bi, bi)

    xs_i = pts_c_ref[ds, :] * scale
    xs_jt = pts_r_ref[...] * scale
    n_i = nrm_c_ref[ds, :]

    n = pts_r_ref.shape[1]
    ones_col = jnp.ones((bi, 1), jnp.float32)
    ones_row = jnp.ones((1, n), jnp.float32)
    r2_col = jnp.sum(xs_i * xs_i, axis=1, keepdims=True)
    r2_row = jnp.sum(xs_jt * xs_jt, axis=0, keepdims=True)

    lhs_d2 = jnp.concatenate([xs_i * -2.0, r2_col, ones_col], axis=1)
    rhs_d2 = jnp.concatenate([xs_jt, ones_row, r2_row], axis=0)
    d2 = jnp.dot(lhs_d2, rhs_d2, preferred_element_type=jnp.float32)
    lhs_t = jnp.concatenate([-n_i, 2.0 * ones_col], axis=1)
    rhs_t = jnp.concatenate([nrm_r_ref[...], ones_row], axis=0)
    t = jnp.dot(lhs_t, rhs_t, preferred_element_type=jnp.float32)

    w = jnp.exp(-(d2 * t) * t)

    nuv_i = nuv9_ref[ds, :]
    rhs_l = jnp.concatenate([xs_jt, ones_row], axis=0)
    f = f_ref[...]
    acc = jnp.dot(w, f, preferred_element_type=jnp.float32) * b2r_ref[...]
    for c in range(cuts):
        mc = (a1_ref[c, 0] * nuv_i[:, 0:3] + a1_ref[c, 1] * nuv_i[:, 3:6]
              + a1_ref[c, 2] * nuv_i[:, 6:9])
        col_c = b1c_ref[c] - jnp.sum(mc * xs_i, axis=1, keepdims=True)
        lc = jnp.dot(jnp.concatenate([mc, col_c], axis=1), rhs_l,
                     preferred_element_type=jnp.float32)
        acc = acc + jnp.dot(w * jnp.maximum(lc, 0.0), f,
                            preferred_element_type=jnp.float32) * a2t_ref[c:c + 1, :]

    z = _leaky(jnp.dot(acc, wo1t_ref[...],
                       preferred_element_type=jnp.float32) + bo1_ref[...])
    z = _leaky(jnp.dot(z, wo2t_ref[...],
                       preferred_element_type=jnp.float32) + bo2_ref[...])
    z_ref[ds, :] = z

    @pl.when(i == n_steps - 1)
    def _finish():
        out_ref[...] = _group_norm(z_ref[...], gout_ref[...], beout_ref[...],
                                   _GROUPS)


def kernel(points, nuv, features, W1, b1, W2, b2, g_in, be_in, A1, B1,
           A2, B2, Wo1, bo1, Wo2, bo2, g_out, be_out):
    n = points.shape[0]
    h = W1.shape[0]
    o = Wo1.shape[0]
    cuts = A1.shape[0]
    n_steps = n // _BI

    normals = nuv[:, 0, :]
    args = (
        points, points.T, normals, normals.T, nuv.reshape(n, 9),
        features, W1.T, b1[None], W2.T, b2[None], g_in[None], be_in[None],
        A1, B1, A2.T, B2[None],
        Wo1.T, bo1[None], Wo2.T, bo2[None], g_out[None], be_out[None],
    )

    def full_spec(a):
        nd = a.ndim
        return pl.BlockSpec(a.shape, lambda i, _nd=nd: (0,) * _nd)

    in_specs = [full_spec(a) for a in args]
    in_specs[12] = pl.BlockSpec(memory_space=pltpu.SMEM)
    in_specs[13] = pl.BlockSpec(memory_space=pltpu.SMEM)

    body = functools.partial(_body, n_steps=n_steps, bi=_BI, cuts=cuts)
    return pl.pallas_call(
        body,
        grid=(n_steps,),
        in_specs=in_specs,
        out_specs=pl.BlockSpec((n, o), lambda i: (0, 0)),
        out_shape=jax.ShapeDtypeStruct((n, o), jnp.float32),
        scratch_shapes=[
            pltpu.VMEM((n, h), jnp.float32),
            pltpu.VMEM((n, o), jnp.float32),
        ],
        compiler_params=pltpu.CompilerParams(
            dimension_semantics=("arbitrary",)),
    )(*args)
